# baseline (speedup 1.0000x reference)
import jax
import jax.numpy as jnp
from jax import lax
from jax.experimental import pallas as pl
from jax.experimental.pallas import tpu as pltpu
from jax.experimental.pallas import tpu_sc as plsc

N = 10000
NP = 10240
E = 320000
D = 128
K = 3
C = 80
NT = 16
NW = 32
EP = 327680
EPT = EP // NW
RPT = EPT // C
NB = 4
BR = 80
_MESH = plsc.VectorSubcoreMesh(core_axis_name="c", subcore_axis_name="s")


def _zero16():
    return jnp.zeros((16,), jnp.float32)


def _deg_kernel_body(dst2_hbm, out_hbm, didx_v, dacc_v):
    c = lax.axis_index("c")
    s = lax.axis_index("s")
    wid = c * NT + s
    ones16 = jnp.full((16,), 1.0, jnp.float32)

    def zi(i, carry):
        dacc_v[pl.ds(i * 16, 16)] = _zero16()
        return carry

    lax.fori_loop(0, NP // 16, zi, 0)
    pltpu.sync_copy(dst2_hbm.at[pl.ds(wid * RPT, RPT), :], didx_v)

    def body(i, carry):
        for j in range(C // 16):
            idx = didx_v[i, pl.ds(j * 16, 16)]
            plsc.addupdate_scatter(dacc_v, [idx], ones16)
        return carry

    lax.fori_loop(0, RPT, body, 0)
    pltpu.sync_copy(dacc_v, out_hbm.at[pl.ds(wid * NP, NP)])


def _sc_degree(dst2):
    f = pl.kernel(
        _deg_kernel_body,
        out_type=jax.ShapeDtypeStruct((NW * NP,), jnp.float32),
        mesh=_MESH,
        scratch_types=[
            pltpu.VMEM((RPT, C), jnp.int32),
            pltpu.VMEM((NP,), jnp.float32),
        ],
        compiler_params=pltpu.CompilerParams(needs_layout_passes=False),
    )
    return f(dst2)


def _hop_kernel_body(src1_hbm, dst1_hbm, tbl, out0, out1,
                     idx_v, didx_v, rows_v, sem, acc_sh):
    c = lax.axis_index("c")
    s = lax.axis_index("s")

    def zi(i, carry):
        for j in range(D // 16):
            rows_v[i, pl.ds(j * 16, 16)] = _zero16()
        return carry

    lax.fori_loop(0, C, zi, 0)
    for j in range(NP // NT // C):
        pltpu.sync_copy(rows_v,
                        acc_sh.at[pl.ds(s * (NP // NT) + j * C, C), :])
    plsc.subcore_barrier()

    base0 = (c * NT + s) * EPT

    def body(i, carry):
        b = base0 + i * C
        pltpu.sync_copy(src1_hbm.at[pl.ds(b, C)], idx_v)
        pltpu.sync_copy(dst1_hbm.at[pl.ds(b, C)], didx_v)
        pltpu.async_copy(tbl.at[idx_v], rows_v, sem).wait()
        pltpu.sync_copy(rows_v, acc_sh.at[didx_v], add=True)
        return carry

    lax.fori_loop(0, EPT // C, body, 0)
    plsc.subcore_barrier()

    row0 = s * (NP // NT)

    @pl.when(c == 0)
    def _():
        pltpu.sync_copy(acc_sh.at[pl.ds(row0, NP // NT), :],
                        out0.at[pl.ds(row0, NP // NT), :])

    @pl.when(c == 1)
    def _():
        pltpu.sync_copy(acc_sh.at[pl.ds(row0, NP // NT), :],
                        out1.at[pl.ds(row0, NP // NT), :])


def _sc_hop(src1, dst1, g):
    f = pl.kernel(
        _hop_kernel_body,
        out_type=(jax.ShapeDtypeStruct((NP, D), jnp.float32),
                  jax.ShapeDtypeStruct((NP, D), jnp.float32)),
        mesh=_MESH,
        scratch_types=[
            pltpu.VMEM((C,), jnp.int32),
            pltpu.VMEM((C,), jnp.int32),
            pltpu.VMEM((C, D), jnp.float32),
            pltpu.SemaphoreType.DMA,
            pltpu.VMEM_SHARED((NP, D), jnp.float32),
        ],
    )
    return f(src1, dst1, g)


def _scale0_body(x_ref, dT_ref, g_ref):
    deg = jnp.sum(dT_ref[...], axis=1, keepdims=True) + 1.0
    g_ref[...] = x_ref[...] * lax.rsqrt(deg)


def _combine_body(p0_ref, p1_ref, g_ref, dT_ref, o_ref):
    deg = jnp.sum(dT_ref[...], axis=1, keepdims=True) + 1.0
    o_ref[...] = (p0_ref[...] + p1_ref[...] + g_ref[...]) / deg


def _final_body(p0_ref, p1_ref, g_ref, dT_ref, w_ref, b_ref, out_ref):
    deg = jnp.sum(dT_ref[...], axis=1, keepdims=True) + 1.0
    h = (p0_ref[...] + p1_ref[...] + g_ref[...]) * lax.rsqrt(deg)
    o = jnp.dot(h, w_ref[...], preferred_element_type=jnp.float32) + b_ref[...]
    m = jnp.max(o, axis=1, keepdims=True)
    lse = jnp.log(jnp.sum(jnp.exp(o - m), axis=1, keepdims=True)) + m
    out_ref[...] = o - lse


def _row_spec(width):
    return pl.BlockSpec((BR, width), lambda i: (i, 0))


def _tc_scale0(x, dT):
    return pl.pallas_call(
        _scale0_body,
        grid=(N // BR,),
        in_specs=[_row_spec(D), _row_spec(NW)],
        out_specs=_row_spec(D),
        out_shape=jax.ShapeDtypeStruct((N, D), jnp.float32),
    )(x, dT)


def _tc_combine(p0, p1, g, dT):
    return pl.pallas_call(
        _combine_body,
        grid=(N // BR,),
        in_specs=[_row_spec(D)] * 3 + [_row_spec(NW)],
        out_specs=_row_spec(D),
        out_shape=jax.ShapeDtypeStruct((N, D), jnp.float32),
    )(p0, p1, g, dT)


def _tc_final(p0, p1, g, dT, W, b):
    return pl.pallas_call(
        _final_body,
        grid=(N // BR,),
        in_specs=[_row_spec(D)] * 3 + [_row_spec(NW)] + [
            pl.BlockSpec((D, D), lambda i: (0, 0)),
            pl.BlockSpec((1, D), lambda i: (0, 0)),
        ],
        out_specs=_row_spec(D),
        out_shape=jax.ShapeDtypeStruct((N, D), jnp.float32),
    )(p0, p1, g, dT, W, b)


def kernel(x, edge_index, W, b):
    pad = EP - E
    src1 = jnp.concatenate(
        [edge_index[0], jnp.zeros((pad,), jnp.int32)])
    dst1 = jnp.concatenate(
        [edge_index[1], jnp.full((pad,), N, jnp.int32)])
    dall = _sc_degree(dst1.reshape(EP // C, C))
    dT = dall.reshape(NW, NP)[:, :N].T
    g = _tc_scale0(x, dT)
    for k in range(K):
        p0, p1 = _sc_hop(src1, dst1, g)
        p0 = p0[:N]
        p1 = p1[:N]
        if k < K - 1:
            g = _tc_combine(p0, p1, g, dT)
    return _tc_final(p0, p1, g, dT, W, b.reshape(1, D))


# speedup vs baseline: 1.0072x; 1.0072x over previous
import jax
import jax.numpy as jnp
from jax import lax
from jax.experimental import pallas as pl
from jax.experimental.pallas import tpu as pltpu
from jax.experimental.pallas import tpu_sc as plsc

N = 10000
NP = 10240
E = 320000
D = 128
K = 3
C = 80
NT = 16
NW = 32
EP = 327680
EPT = EP // NW
RPT = EPT // C
NB = 4
BR = 80
_MESH = plsc.VectorSubcoreMesh(core_axis_name="c", subcore_axis_name="s")


def _zero16():
    return jnp.zeros((16,), jnp.float32)


def _deg_kernel_body(dst2_hbm, out_hbm, didx_v, dacc_v):
    c = lax.axis_index("c")
    s = lax.axis_index("s")
    wid = c * NT + s
    ones16 = jnp.full((16,), 1.0, jnp.float32)

    def zi(i, carry):
        dacc_v[pl.ds(i * 16, 16)] = _zero16()
        return carry

    lax.fori_loop(0, NP // 16, zi, 0)
    pltpu.sync_copy(dst2_hbm.at[pl.ds(wid * RPT, RPT), :], didx_v)

    def body(i, carry):
        for j in range(C // 16):
            idx = didx_v[i, pl.ds(j * 16, 16)]
            plsc.addupdate_scatter(dacc_v, [idx], ones16)
        return carry

    lax.fori_loop(0, RPT, body, 0)
    pltpu.sync_copy(dacc_v, out_hbm.at[pl.ds(wid * NP, NP)])


def _sc_degree(dst2):
    f = pl.kernel(
        _deg_kernel_body,
        out_type=jax.ShapeDtypeStruct((NW * NP,), jnp.float32),
        mesh=_MESH,
        scratch_types=[
            pltpu.VMEM((RPT, C), jnp.int32),
            pltpu.VMEM((NP,), jnp.float32),
        ],
        compiler_params=pltpu.CompilerParams(needs_layout_passes=False),
    )
    return f(dst2)


def _hop_kernel_body(src1_hbm, dst1_hbm, tbl, out0, out1,
                     idx_v, didx_v, rows_v, sem, acc_sh):
    c = lax.axis_index("c")
    s = lax.axis_index("s")

    def zi(i, carry):
        for j in range(D // 16):
            rows_v[i, pl.ds(j * 16, 16)] = _zero16()
        return carry

    lax.fori_loop(0, C, zi, 0)
    for j in range(NP // NT // C):
        pltpu.sync_copy(rows_v,
                        acc_sh.at[pl.ds(s * (NP // NT) + j * C, C), :])
    plsc.subcore_barrier()

    base0 = (c * NT + s) * EPT

    def body(i, carry):
        b = base0 + i * C
        pltpu.sync_copy(src1_hbm.at[pl.ds(b, C)], idx_v)
        pltpu.sync_copy(dst1_hbm.at[pl.ds(b, C)], didx_v)
        pltpu.async_copy(tbl.at[idx_v], rows_v, sem).wait()
        pltpu.sync_copy(rows_v, acc_sh.at[didx_v], add=True)
        return carry

    lax.fori_loop(0, EPT // C, body, 0)
    plsc.subcore_barrier()

    row0 = s * (NP // NT)

    @pl.when(c == 0)
    def _():
        pltpu.sync_copy(acc_sh.at[pl.ds(row0, NP // NT), :],
                        out0.at[pl.ds(row0, NP // NT), :])

    @pl.when(c == 1)
    def _():
        pltpu.sync_copy(acc_sh.at[pl.ds(row0, NP // NT), :],
                        out1.at[pl.ds(row0, NP // NT), :])


def _sc_hop(src1, dst1, g):
    f = pl.kernel(
        _hop_kernel_body,
        out_type=(jax.ShapeDtypeStruct((NP, D), jnp.float32),
                  jax.ShapeDtypeStruct((NP, D), jnp.float32)),
        mesh=_MESH,
        scratch_types=[
            pltpu.VMEM((C,), jnp.int32),
            pltpu.VMEM((C,), jnp.int32),
            pltpu.VMEM((C, D), jnp.float32),
            pltpu.SemaphoreType.DMA,
            pltpu.VMEM_SHARED((NP, D), jnp.float32),
        ],
    )
    return f(src1, dst1, g)


def _scale0_body(x_ref, dT_ref, g_ref):
    deg = jnp.sum(dT_ref[...], axis=1, keepdims=True) + 1.0
    g_ref[...] = x_ref[...] * lax.rsqrt(deg)


def _combine_body(p0_ref, p1_ref, g_ref, dT_ref, o_ref):
    deg = jnp.sum(dT_ref[...], axis=1, keepdims=True) + 1.0
    o_ref[...] = (p0_ref[...] + p1_ref[...] + g_ref[...]) / deg


def _final_body(p0_ref, p1_ref, g_ref, dT_ref, w_ref, b_ref, out_ref):
    deg = jnp.sum(dT_ref[...], axis=1, keepdims=True) + 1.0
    h = (p0_ref[...] + p1_ref[...] + g_ref[...]) * lax.rsqrt(deg)
    o = jnp.dot(h, w_ref[...], preferred_element_type=jnp.float32) + b_ref[...]
    m = jnp.max(o, axis=1, keepdims=True)
    lse = jnp.log(jnp.sum(jnp.exp(o - m), axis=1, keepdims=True)) + m
    out_ref[...] = o - lse


def _row_spec(width):
    return pl.BlockSpec((BR, width), lambda i: (i, 0))


def _tc_scale0(x, dT):
    return pl.pallas_call(
        _scale0_body,
        grid=(N // BR,),
        in_specs=[_row_spec(D), _row_spec(NW)],
        out_specs=_row_spec(D),
        out_shape=jax.ShapeDtypeStruct((N, D), jnp.float32),
    )(x, dT)


def _tc_combine(p0, p1, g, dT):
    return pl.pallas_call(
        _combine_body,
        grid=(N // BR,),
        in_specs=[_row_spec(D)] * 3 + [_row_spec(NW)],
        out_specs=_row_spec(D),
        out_shape=jax.ShapeDtypeStruct((N, D), jnp.float32),
    )(p0, p1, g, dT)


def _tc_final(p0, p1, g, dT, W, b):
    return pl.pallas_call(
        _final_body,
        grid=(N // BR,),
        in_specs=[_row_spec(D)] * 3 + [_row_spec(NW)] + [
            pl.BlockSpec((D, D), lambda i: (0, 0)),
            pl.BlockSpec((1, D), lambda i: (0, 0)),
        ],
        out_specs=_row_spec(D),
        out_shape=jax.ShapeDtypeStruct((N, D), jnp.float32),
    )(p0, p1, g, dT, W, b)


def kernel(x, edge_index, W, b):
    pad = EP - E
    src1 = jnp.concatenate(
        [edge_index[0], jnp.zeros((pad,), jnp.int32)])
    dst1 = jnp.concatenate(
        [edge_index[1],
         N + (jnp.arange(pad, dtype=jnp.int32) % (NP - N))])
    dall = _sc_degree(dst1.reshape(EP // C, C))
    dT = dall.reshape(NW, NP)[:, :N].T
    g = _tc_scale0(x, dT)
    for k in range(K):
        p0, p1 = _sc_hop(src1, dst1, g)
        p0 = p0[:N]
        p1 = p1[:N]
        if k < K - 1:
            g = _tc_combine(p0, p1, g, dT)
    return _tc_final(p0, p1, g, dT, W, b.reshape(1, D))


# speedup vs baseline: 1.6188x; 1.6072x over previous
import jax
import jax.numpy as jnp
from jax import lax
from jax.experimental import pallas as pl
from jax.experimental.pallas import tpu as pltpu
from jax.experimental.pallas import tpu_sc as plsc

N = 10000
NP = 10240
E = 320000
D = 128
K = 3
C = 80
NT = 16
NW = 32
EP = 327680
EPT = EP // NW
RPT = EPT // C
NB = 4
BR = 80
_MESH = plsc.VectorSubcoreMesh(core_axis_name="c", subcore_axis_name="s")


def _zero16():
    return jnp.zeros((16,), jnp.float32)


def _deg_kernel_body(dst_hbm, out_hbm, didx_v, dacc_v):
    c = lax.axis_index("c")
    s = lax.axis_index("s")
    wid = c * NT + s
    ones16 = jnp.full((16,), 1.0, jnp.float32)

    def zi(i, carry):
        dacc_v[pl.ds(i * 16, 16)] = _zero16()
        return carry

    lax.fori_loop(0, NP // 16, zi, 0)

    base0 = wid * (E // NW)

    def body(i, carry):
        pltpu.sync_copy(dst_hbm.at[pl.ds(base0 + i * C, C)], didx_v)
        for j in range(C // 16):
            idx = didx_v[pl.ds(j * 16, 16)]
            plsc.addupdate_scatter(dacc_v, [idx], ones16)
        return carry

    lax.fori_loop(0, E // NW // C, body, 0)
    pltpu.sync_copy(dacc_v, out_hbm.at[pl.ds(wid * NP, NP)])


def _sc_degree(dst):
    f = pl.kernel(
        _deg_kernel_body,
        out_type=jax.ShapeDtypeStruct((NW * NP,), jnp.float32),
        mesh=_MESH,
        scratch_types=[
            pltpu.VMEM((C,), jnp.int32),
            pltpu.VMEM((NP,), jnp.float32),
        ],
        compiler_params=pltpu.CompilerParams(needs_layout_passes=False),
    )
    return f(dst)


def _hop_kernel_body(src1_hbm, dst1_hbm, tbl, out0, out1,
                     idx_v, didx_v, rows_v, sem, acc_sh):
    c = lax.axis_index("c")
    s = lax.axis_index("s")

    def zi(i, carry):
        for j in range(D // 16):
            rows_v[i, pl.ds(j * 16, 16)] = _zero16()
        return carry

    lax.fori_loop(0, C, zi, 0)
    for j in range(NP // NT // C):
        pltpu.sync_copy(rows_v,
                        acc_sh.at[pl.ds(s * (NP // NT) + j * C, C), :])
    plsc.subcore_barrier()

    base0 = (c * NT + s) * (E // NW)

    def body(i, carry):
        b = base0 + i * C
        pltpu.sync_copy(src1_hbm.at[pl.ds(b, C)], idx_v)
        pltpu.sync_copy(dst1_hbm.at[pl.ds(b, C)], didx_v)
        pltpu.async_copy(tbl.at[idx_v], rows_v, sem).wait()
        pltpu.sync_copy(rows_v, acc_sh.at[didx_v], add=True)
        return carry

    lax.fori_loop(0, E // NW // C, body, 0)
    plsc.subcore_barrier()

    row0 = s * (NP // NT)

    @pl.when(c == 0)
    def _():
        pltpu.sync_copy(acc_sh.at[pl.ds(row0, NP // NT), :],
                        out0.at[pl.ds(row0, NP // NT), :])

    @pl.when(c == 1)
    def _():
        pltpu.sync_copy(acc_sh.at[pl.ds(row0, NP // NT), :],
                        out1.at[pl.ds(row0, NP // NT), :])


def _sc_hop(src1, dst1, g):
    f = pl.kernel(
        _hop_kernel_body,
        out_type=(jax.ShapeDtypeStruct((NP, D), jnp.float32),
                  jax.ShapeDtypeStruct((NP, D), jnp.float32)),
        mesh=_MESH,
        scratch_types=[
            pltpu.VMEM((C,), jnp.int32),
            pltpu.VMEM((C,), jnp.int32),
            pltpu.VMEM((C, D), jnp.float32),
            pltpu.SemaphoreType.DMA,
            pltpu.VMEM_SHARED((NP, D), jnp.float32),
        ],
    )
    return f(src1, dst1, g)


def _scale0_body(x_ref, dT_ref, g_ref):
    deg = jnp.sum(dT_ref[...], axis=1, keepdims=True) + 1.0
    g_ref[...] = x_ref[...] * lax.rsqrt(deg)


def _combine_body(p0_ref, p1_ref, g_ref, dT_ref, o_ref):
    deg = jnp.sum(dT_ref[...], axis=1, keepdims=True) + 1.0
    o_ref[...] = (p0_ref[...] + p1_ref[...] + g_ref[...]) / deg


def _final_body(p0_ref, p1_ref, g_ref, dT_ref, w_ref, b_ref, out_ref):
    deg = jnp.sum(dT_ref[...], axis=1, keepdims=True) + 1.0
    h = (p0_ref[...] + p1_ref[...] + g_ref[...]) * lax.rsqrt(deg)
    o = jnp.dot(h, w_ref[...], preferred_element_type=jnp.float32) + b_ref[...]
    m = jnp.max(o, axis=1, keepdims=True)
    lse = jnp.log(jnp.sum(jnp.exp(o - m), axis=1, keepdims=True)) + m
    out_ref[...] = o - lse


def _row_spec(width):
    return pl.BlockSpec((BR, width), lambda i: (i, 0))


def _tc_scale0(x, dT):
    return pl.pallas_call(
        _scale0_body,
        grid=(N // BR,),
        in_specs=[_row_spec(D), _row_spec(NW)],
        out_specs=_row_spec(D),
        out_shape=jax.ShapeDtypeStruct((N, D), jnp.float32),
    )(x, dT)


def _tc_combine(p0, p1, g, dT):
    return pl.pallas_call(
        _combine_body,
        grid=(N // BR,),
        in_specs=[_row_spec(D)] * 3 + [_row_spec(NW)],
        out_specs=_row_spec(D),
        out_shape=jax.ShapeDtypeStruct((N, D), jnp.float32),
    )(p0, p1, g, dT)


def _tc_final(p0, p1, g, dT, W, b):
    return pl.pallas_call(
        _final_body,
        grid=(N // BR,),
        in_specs=[_row_spec(D)] * 3 + [_row_spec(NW)] + [
            pl.BlockSpec((D, D), lambda i: (0, 0)),
            pl.BlockSpec((1, D), lambda i: (0, 0)),
        ],
        out_specs=_row_spec(D),
        out_shape=jax.ShapeDtypeStruct((N, D), jnp.float32),
    )(p0, p1, g, dT, W, b)


def kernel(x, edge_index, W, b):
    src1 = edge_index[0]
    dst1 = edge_index[1]
    dall = _sc_degree(dst1)
    dT = dall.reshape(NW, NP)[:, :N].T
    g = _tc_scale0(x, dT)
    for k in range(K):
        p0, p1 = _sc_hop(src1, dst1, g)
        p0 = p0[:N]
        p1 = p1[:N]
        if k < K - 1:
            g = _tc_combine(p0, p1, g, dT)
    return _tc_final(p0, p1, g, dT, W, b.reshape(1, D))
